# baseline (speedup 1.0000x reference)
import functools
import math

import jax
import jax.numpy as jnp
from jax import lax
from jax.experimental import pallas as pl
from jax.experimental.pallas import tpu as pltpu
from jax.experimental.pallas import tpu_sc as plsc

_TOP_K = 8
_PAD_K = 16


def _mv_body(q_ref, k_ref, mv_ref, *, scale):
    h = pl.program_id(1)

    @pl.when(h == 0)
    def _():
        mv_ref[...] = jnp.zeros_like(mv_ref)

    q = q_ref[0, 0]
    k = k_ref[0, 0]
    sq = jnp.sum(q, axis=0)
    sk = jnp.sum(k, axis=0)
    mv_ref[0, 0, :] += sq * sk * scale


def _mean_value(queries, keys):
    B, H, L, E = queries.shape
    return pl.pallas_call(
        functools.partial(_mv_body, scale=1.0 / (H * L)),
        grid=(B, H),
        in_specs=[
            pl.BlockSpec((1, 1, L, E), lambda b, h: (b, h, 0, 0)),
            pl.BlockSpec((1, 1, L, E), lambda b, h: (b, h, 0, 0)),
        ],
        out_specs=pl.BlockSpec((1, 1, E), lambda b, h: (b, 0, 0)),
        out_shape=jax.ShapeDtypeStruct((B, 1, E), jnp.float32),
    )(queries, keys)


def _select_sc(mean_value):
    B, _, E = mean_value.shape
    mesh = plsc.ScalarSubcoreMesh(axis_name="c", num_cores=2)

    @functools.partial(
        pl.kernel,
        out_type=[
            jax.ShapeDtypeStruct((1, _PAD_K), jnp.int32),
            jax.ShapeDtypeStruct((B, _TOP_K), jnp.float32),
        ],
        mesh=mesh,
        scratch_types=[
            pltpu.SMEM((B, 1, E), jnp.float32),
            pltpu.SMEM((E,), jnp.float32),
            pltpu.SMEM((1, _PAD_K), jnp.int32),
            pltpu.SMEM((B, _TOP_K), jnp.float32),
            pltpu.SemaphoreType.DMA,
        ],
    )
    def sel(mv_hbm, idx_hbm, w_hbm, mv_s, gm_s, idx_s, w_s, sem):
        core = lax.axis_index("c")

        @pl.when(core == 0)
        def _():
            pltpu.async_copy(mv_hbm, mv_s, sem).wait()

            @pl.loop(0, E)
            def _(e):
                acc = mv_s[0, 0, e]
                for b in range(1, B):
                    acc = acc + mv_s[b, 0, e]
                gm_s[e] = acc * (1.0 / B)

            @pl.loop(0, _PAD_K)
            def _(i):
                idx_s[0, i] = 0

            for i in range(_TOP_K):
                def body(e, carry):
                    bv, bi = carry
                    v = gm_s[e]
                    better = v > bv
                    return (
                        jnp.where(better, v, bv),
                        jnp.where(better, e, bi),
                    )

                bv, bi = lax.fori_loop(
                    0, E, body, (jnp.float32(-jnp.inf), jnp.int32(0))
                )
                idx_s[0, i] = bi
                for b in range(B):
                    w_s[b, i] = mv_s[b, 0, bi]
                gm_s[bi] = -jnp.inf

            pltpu.async_copy(idx_s, idx_hbm, sem).wait()
            pltpu.async_copy(w_s, w_hbm, sem).wait()

    return sel(mean_value)


def _select_body(mv_ref, idx_ref, w_ref):
    B, _, E = mv_ref.shape
    mv = mv_ref[:, 0, :]
    gm = jnp.mean(mv, axis=0, keepdims=True)
    iota_e = lax.broadcasted_iota(jnp.int32, (1, E), 1)
    iota_k = lax.broadcasted_iota(jnp.int32, (1, _PAD_K), 1)
    iota_kb = lax.broadcasted_iota(jnp.int32, (B, _TOP_K), 1)
    idx_row = jnp.zeros((1, _PAD_K), jnp.int32)
    w = jnp.zeros((B, _TOP_K), jnp.float32)
    for i in range(_TOP_K):
        m = jnp.max(gm)
        pos = jnp.min(jnp.where(gm == m, iota_e, E))
        onehot = iota_e == pos
        idx_row = idx_row + pos * (iota_k == i).astype(jnp.int32)
        wcol = jnp.sum(jnp.where(onehot, mv, 0.0), axis=1, keepdims=True)
        w = w + wcol * (iota_kb == i).astype(jnp.float32)
        gm = jnp.where(onehot, -jnp.inf, gm)
    idx_ref[...] = idx_row
    w_ref[...] = w


def _select(mean_value):
    B, _, E = mean_value.shape
    return pl.pallas_call(
        _select_body,
        in_specs=[pl.BlockSpec((B, 1, E), lambda: (0, 0, 0))],
        out_specs=[
            pl.BlockSpec((1, _PAD_K), lambda: (0, 0)),
            pl.BlockSpec((B, _TOP_K), lambda: (0, 0)),
        ],
        out_shape=[
            jax.ShapeDtypeStruct((1, _PAD_K), jnp.int32),
            jax.ShapeDtypeStruct((B, _TOP_K), jnp.float32),
        ],
    )(mean_value)


_WRAP = 128


def _agg_body(idx_ref, w_ref, v_ref, out_ref, scr_ref):
    b = pl.program_id(0)
    L = v_ref.shape[2]

    v = v_ref[0, 0]
    scr_ref[:_WRAP, :] = v[L - _WRAP :, :]
    scr_ref[_WRAP:, :] = v[:_WRAP, :]

    row = w_ref[pl.ds(b, 1), :]
    row = row - jnp.max(row, axis=1, keepdims=True)
    e = jnp.exp(row)
    sm = e / jnp.sum(e, axis=1, keepdims=True)

    acc = None
    tail = None
    for i in range(_TOP_K):
        d = idx_ref[0, i]
        wi = sm[0:1, i : i + 1]
        term = v_ref[0, 0, pl.ds(d, L - _WRAP), :] * wi
        acc = term if acc is None else acc + term
        tterm = scr_ref[pl.ds(d, _WRAP), :] * wi
        tail = tterm if tail is None else tail + tterm
    out_ref[0, 0, : L - _WRAP, :] = acc
    out_ref[0, 0, L - _WRAP :, :] = tail


def _aggregate(values, idx, w):
    B, H, L, E = values.shape
    return pl.pallas_call(
        _agg_body,
        grid=(B, H),
        in_specs=[
            pl.BlockSpec(memory_space=pltpu.SMEM),
            pl.BlockSpec((B, _TOP_K), lambda b, h: (0, 0)),
            pl.BlockSpec((1, 1, L, E), lambda b, h: (b, h, 0, 0)),
        ],
        out_specs=pl.BlockSpec((1, 1, L, E), lambda b, h: (b, h, 0, 0)),
        out_shape=jax.ShapeDtypeStruct((B, H, L, E), jnp.float32),
        scratch_shapes=[pltpu.VMEM((2 * _WRAP, E), jnp.float32)],
    )(idx, w, values)


def kernel(queries, keys, values):
    mv = _mean_value(queries, keys)
    idx, w = _select_sc(mv)
    return _aggregate(values, idx, w)


# speedup vs baseline: 1.2159x; 1.2159x over previous
import functools
import math

import jax
import jax.numpy as jnp
from jax import lax
from jax.experimental import pallas as pl
from jax.experimental.pallas import tpu as pltpu
from jax.experimental.pallas import tpu_sc as plsc

_TOP_K = 8
_PAD_K = 16


def _mv_body(q_ref, k_ref, mv_ref, *, scale):
    h = pl.program_id(1)

    @pl.when(h == 0)
    def _():
        mv_ref[...] = jnp.zeros_like(mv_ref)

    q = q_ref[0, 0]
    k = k_ref[0, 0]
    sq = jnp.sum(q, axis=0)
    sk = jnp.sum(k, axis=0)
    mv_ref[0, 0, :] += sq * sk * scale


def _mean_value(queries, keys):
    B, H, L, E = queries.shape
    return pl.pallas_call(
        functools.partial(_mv_body, scale=1.0 / (H * L)),
        grid=(B, H),
        in_specs=[
            pl.BlockSpec((1, 1, L, E), lambda b, h: (b, h, 0, 0)),
            pl.BlockSpec((1, 1, L, E), lambda b, h: (b, h, 0, 0)),
        ],
        out_specs=pl.BlockSpec((1, 1, E), lambda b, h: (b, 0, 0)),
        out_shape=jax.ShapeDtypeStruct((B, 1, E), jnp.float32),
    )(queries, keys)


def _select_sc(mean_value):
    B, _, E = mean_value.shape
    mesh = plsc.ScalarSubcoreMesh(axis_name="c", num_cores=2)

    @functools.partial(
        pl.kernel,
        out_type=[
            jax.ShapeDtypeStruct((1, _PAD_K), jnp.int32),
            jax.ShapeDtypeStruct((B, _TOP_K), jnp.float32),
        ],
        mesh=mesh,
        scratch_types=[
            pltpu.SMEM((B, 1, E), jnp.float32),
            pltpu.SMEM((E,), jnp.float32),
            pltpu.SMEM((1, _PAD_K), jnp.int32),
            pltpu.SMEM((B, _TOP_K), jnp.float32),
            pltpu.SemaphoreType.DMA,
        ],
    )
    def sel(mv_hbm, idx_hbm, w_hbm, mv_s, gm_s, idx_s, w_s, sem):
        core = lax.axis_index("c")

        @pl.when(core == 0)
        def _():
            pltpu.async_copy(mv_hbm, mv_s, sem).wait()

            @pl.loop(0, E)
            def _(e):
                acc = mv_s[0, 0, e]
                for b in range(1, B):
                    acc = acc + mv_s[b, 0, e]
                gm_s[e] = acc * (1.0 / B)

            @pl.loop(0, _PAD_K)
            def _(i):
                idx_s[0, i] = 0

            for i in range(_TOP_K):
                def body(e, carry):
                    bv, bi = carry
                    v = gm_s[e]
                    better = v > bv
                    return (
                        jnp.where(better, v, bv),
                        jnp.where(better, e, bi),
                    )

                bv, bi = lax.fori_loop(
                    0, E, body, (jnp.float32(-jnp.inf), jnp.int32(0))
                )
                idx_s[0, i] = bi
                for b in range(B):
                    w_s[b, i] = mv_s[b, 0, bi]
                gm_s[bi] = -jnp.inf

            pltpu.async_copy(idx_s, idx_hbm, sem).wait()
            pltpu.async_copy(w_s, w_hbm, sem).wait()

    return sel(mean_value)


def _select_body(mv_ref, idx_ref, w_ref):
    B, _, E = mv_ref.shape
    mv = mv_ref[:, 0, :]
    gm = jnp.mean(mv, axis=0, keepdims=True)
    iota_e = lax.broadcasted_iota(jnp.int32, (1, E), 1)
    iota_k = lax.broadcasted_iota(jnp.int32, (1, _PAD_K), 1)
    iota_kb = lax.broadcasted_iota(jnp.int32, (B, _TOP_K), 1)
    idx_row = jnp.zeros((1, _PAD_K), jnp.int32)
    w = jnp.zeros((B, _TOP_K), jnp.float32)
    for i in range(_TOP_K):
        m = jnp.max(gm)
        pos = jnp.min(jnp.where(gm == m, iota_e, E))
        onehot = iota_e == pos
        idx_row = idx_row + pos * (iota_k == i).astype(jnp.int32)
        wcol = jnp.sum(jnp.where(onehot, mv, 0.0), axis=1, keepdims=True)
        w = w + wcol * (iota_kb == i).astype(jnp.float32)
        gm = jnp.where(onehot, -jnp.inf, gm)
    idx_ref[...] = idx_row
    w_ref[...] = w


def _select(mean_value):
    B, _, E = mean_value.shape
    return pl.pallas_call(
        _select_body,
        in_specs=[pl.BlockSpec((B, 1, E), lambda: (0, 0, 0))],
        out_specs=[
            pl.BlockSpec((1, _PAD_K), lambda: (0, 0)),
            pl.BlockSpec((B, _TOP_K), lambda: (0, 0)),
        ],
        out_shape=[
            jax.ShapeDtypeStruct((1, _PAD_K), jnp.int32),
            jax.ShapeDtypeStruct((B, _TOP_K), jnp.float32),
        ],
    )(mean_value)


_WRAP = 128


def _agg_body(idx_ref, w_ref, v_ref, out_ref, scr_ref):
    b = pl.program_id(0)
    L = v_ref.shape[2]

    v = v_ref[0, 0]
    scr_ref[:_WRAP, :] = v[L - _WRAP :, :]
    scr_ref[_WRAP:, :] = v[:_WRAP, :]

    row = w_ref[pl.ds(b, 1), :]
    row = row - jnp.max(row, axis=1, keepdims=True)
    e = jnp.exp(row)
    sm = e / jnp.sum(e, axis=1, keepdims=True)

    ds = [idx_ref[0, i] for i in range(_TOP_K)]
    TILE = _WRAP
    for t in range((L - _WRAP) // TILE):
        acc = None
        for i in range(_TOP_K):
            term = v_ref[0, 0, pl.ds(ds[i] + t * TILE, TILE), :] * sm[0:1, i : i + 1]
            acc = term if acc is None else acc + term
        out_ref[0, 0, t * TILE : (t + 1) * TILE, :] = acc
    tail = None
    for i in range(_TOP_K):
        tterm = scr_ref[pl.ds(ds[i], _WRAP), :] * sm[0:1, i : i + 1]
        tail = tterm if tail is None else tail + tterm
    out_ref[0, 0, L - _WRAP :, :] = tail


def _aggregate(values, idx, w):
    B, H, L, E = values.shape
    return pl.pallas_call(
        _agg_body,
        grid=(B, H),
        in_specs=[
            pl.BlockSpec(memory_space=pltpu.SMEM),
            pl.BlockSpec((B, _TOP_K), lambda b, h: (0, 0)),
            pl.BlockSpec((1, 1, L, E), lambda b, h: (b, h, 0, 0)),
        ],
        out_specs=pl.BlockSpec((1, 1, L, E), lambda b, h: (b, h, 0, 0)),
        out_shape=jax.ShapeDtypeStruct((B, H, L, E), jnp.float32),
        scratch_shapes=[pltpu.VMEM((2 * _WRAP, E), jnp.float32)],
    )(idx, w, values)


def kernel(queries, keys, values):
    mv = _mean_value(queries, keys)
    idx, w = _select_sc(mv)
    return _aggregate(values, idx, w)


# speedup vs baseline: 1.4270x; 1.1737x over previous
import functools
import math

import jax
import jax.numpy as jnp
from jax import lax
from jax.experimental import pallas as pl
from jax.experimental.pallas import tpu as pltpu
from jax.experimental.pallas import tpu_sc as plsc

_TOP_K = 8
_PAD_K = 16


_HB = 4


def _mv_body(q_ref, k_ref, mv_ref, *, scale):
    h = pl.program_id(1)

    @pl.when(h == 0)
    def _():
        mv_ref[...] = jnp.zeros_like(mv_ref)

    acc = None
    for j in range(_HB):
        sq = jnp.sum(q_ref[0, j], axis=0)
        sk = jnp.sum(k_ref[0, j], axis=0)
        t = sq * sk
        acc = t if acc is None else acc + t
    mv_ref[0, 0, :] += acc * scale


def _mean_value(queries, keys):
    B, H, L, E = queries.shape
    return pl.pallas_call(
        functools.partial(_mv_body, scale=1.0 / (H * L)),
        grid=(B, H // _HB),
        in_specs=[
            pl.BlockSpec((1, _HB, L, E), lambda b, h: (b, h, 0, 0)),
            pl.BlockSpec((1, _HB, L, E), lambda b, h: (b, h, 0, 0)),
        ],
        out_specs=pl.BlockSpec((1, 1, E), lambda b, h: (b, 0, 0)),
        out_shape=jax.ShapeDtypeStruct((B, 1, E), jnp.float32),
    )(queries, keys)


def _select_sc(mean_value):
    B, _, E = mean_value.shape
    mesh = plsc.ScalarSubcoreMesh(axis_name="c", num_cores=2)

    @functools.partial(
        pl.kernel,
        out_type=[
            jax.ShapeDtypeStruct((1, _PAD_K), jnp.int32),
            jax.ShapeDtypeStruct((B, _TOP_K), jnp.float32),
        ],
        mesh=mesh,
        scratch_types=[
            pltpu.SMEM((B, 1, E), jnp.float32),
            pltpu.SMEM((E,), jnp.float32),
            pltpu.SMEM((1, _PAD_K), jnp.int32),
            pltpu.SMEM((B, _TOP_K), jnp.float32),
            pltpu.SemaphoreType.DMA,
        ],
    )
    def sel(mv_hbm, idx_hbm, w_hbm, mv_s, gm_s, idx_s, w_s, sem):
        core = lax.axis_index("c")

        @pl.when(core == 0)
        def _():
            pltpu.async_copy(mv_hbm, mv_s, sem).wait()

            @pl.loop(0, E)
            def _(e):
                acc = mv_s[0, 0, e]
                for b in range(1, B):
                    acc = acc + mv_s[b, 0, e]
                gm_s[e] = acc * (1.0 / B)

            @pl.loop(0, _PAD_K)
            def _(i):
                idx_s[0, i] = 0

            for i in range(_TOP_K):
                def body(e, carry):
                    bv, bi = carry
                    v = gm_s[e]
                    better = v > bv
                    return (
                        jnp.where(better, v, bv),
                        jnp.where(better, e, bi),
                    )

                bv, bi = lax.fori_loop(
                    0, E, body, (jnp.float32(-jnp.inf), jnp.int32(0))
                )
                idx_s[0, i] = bi
                for b in range(B):
                    w_s[b, i] = mv_s[b, 0, bi]
                gm_s[bi] = -jnp.inf

            pltpu.async_copy(idx_s, idx_hbm, sem).wait()
            pltpu.async_copy(w_s, w_hbm, sem).wait()

    return sel(mean_value)


def _select_body(mv_ref, idx_ref, w_ref):
    B, _, E = mv_ref.shape
    mv = mv_ref[:, 0, :]
    gm = jnp.mean(mv, axis=0, keepdims=True)
    iota_e = lax.broadcasted_iota(jnp.int32, (1, E), 1)
    iota_k = lax.broadcasted_iota(jnp.int32, (1, _PAD_K), 1)
    iota_kb = lax.broadcasted_iota(jnp.int32, (B, _TOP_K), 1)
    idx_row = jnp.zeros((1, _PAD_K), jnp.int32)
    w = jnp.zeros((B, _TOP_K), jnp.float32)
    for i in range(_TOP_K):
        m = jnp.max(gm)
        pos = jnp.min(jnp.where(gm == m, iota_e, E))
        onehot = iota_e == pos
        idx_row = idx_row + pos * (iota_k == i).astype(jnp.int32)
        wcol = jnp.sum(jnp.where(onehot, mv, 0.0), axis=1, keepdims=True)
        w = w + wcol * (iota_kb == i).astype(jnp.float32)
        gm = jnp.where(onehot, -jnp.inf, gm)
    idx_ref[...] = idx_row
    w_ref[...] = w


def _select(mean_value):
    B, _, E = mean_value.shape
    return pl.pallas_call(
        _select_body,
        in_specs=[pl.BlockSpec((B, 1, E), lambda: (0, 0, 0))],
        out_specs=[
            pl.BlockSpec((1, _PAD_K), lambda: (0, 0)),
            pl.BlockSpec((B, _TOP_K), lambda: (0, 0)),
        ],
        out_shape=[
            jax.ShapeDtypeStruct((1, _PAD_K), jnp.int32),
            jax.ShapeDtypeStruct((B, _TOP_K), jnp.float32),
        ],
    )(mean_value)


_WRAP = 128


def _agg_body(idx_ref, w_ref, v_ref, out_ref, scr_ref):
    b = pl.program_id(0)
    L = v_ref.shape[2]

    row = w_ref[pl.ds(b, 1), :]
    row = row - jnp.max(row, axis=1, keepdims=True)
    e = jnp.exp(row)
    sm = e / jnp.sum(e, axis=1, keepdims=True)

    ds = [idx_ref[0, i] for i in range(_TOP_K)]
    TILE = _WRAP
    for j in range(_HB):
        v = v_ref[0, j]
        scr_ref[:_WRAP, :] = v[L - _WRAP :, :]
        scr_ref[_WRAP:, :] = v[:_WRAP, :]
        for t in range((L - _WRAP) // TILE):
            acc = None
            for i in range(_TOP_K):
                term = (
                    v_ref[0, j, pl.ds(ds[i] + t * TILE, TILE), :]
                    * sm[0:1, i : i + 1]
                )
                acc = term if acc is None else acc + term
            out_ref[0, j, t * TILE : (t + 1) * TILE, :] = acc
        tail = None
        for i in range(_TOP_K):
            tterm = scr_ref[pl.ds(ds[i], _WRAP), :] * sm[0:1, i : i + 1]
            tail = tterm if tail is None else tail + tterm
        out_ref[0, j, L - _WRAP :, :] = tail


def _aggregate(values, idx, w):
    B, H, L, E = values.shape
    return pl.pallas_call(
        _agg_body,
        grid=(B, H // _HB),
        in_specs=[
            pl.BlockSpec(memory_space=pltpu.SMEM),
            pl.BlockSpec((B, _TOP_K), lambda b, h: (0, 0)),
            pl.BlockSpec((1, _HB, L, E), lambda b, h: (b, h, 0, 0)),
        ],
        out_specs=pl.BlockSpec((1, _HB, L, E), lambda b, h: (b, h, 0, 0)),
        out_shape=jax.ShapeDtypeStruct((B, H, L, E), jnp.float32),
        scratch_shapes=[pltpu.VMEM((2 * _WRAP, E), jnp.float32)],
    )(idx, w, values)


def kernel(queries, keys, values):
    mv = _mean_value(queries, keys)
    idx, w = _select_sc(mv)
    return _aggregate(values, idx, w)


# speedup vs baseline: 1.4365x; 1.0066x over previous
import functools
import math

import jax
import jax.numpy as jnp
from jax import lax
from jax.experimental import pallas as pl
from jax.experimental.pallas import tpu as pltpu
from jax.experimental.pallas import tpu_sc as plsc

_TOP_K = 8
_PAD_K = 16


_HB = 4


def _mv_body(q_ref, k_ref, mv_ref, *, scale):
    h = pl.program_id(1)

    @pl.when(h == 0)
    def _():
        mv_ref[...] = jnp.zeros_like(mv_ref)

    acc = None
    for j in range(_HB):
        sq = jnp.sum(q_ref[0, j], axis=0)
        sk = jnp.sum(k_ref[0, j], axis=0)
        t = sq * sk
        acc = t if acc is None else acc + t
    mv_ref[0, 0, :] += acc * scale


def _mean_value(queries, keys):
    B, H, L, E = queries.shape
    return pl.pallas_call(
        functools.partial(_mv_body, scale=1.0 / (H * L)),
        grid=(B, H // _HB),
        in_specs=[
            pl.BlockSpec((1, _HB, L, E), lambda b, h: (b, h, 0, 0)),
            pl.BlockSpec((1, _HB, L, E), lambda b, h: (b, h, 0, 0)),
        ],
        out_specs=pl.BlockSpec((1, 1, E), lambda b, h: (b, 0, 0)),
        out_shape=jax.ShapeDtypeStruct((B, 1, E), jnp.float32),
    )(queries, keys)


def _select_sc(mean_value):
    B, _, E = mean_value.shape
    mesh = plsc.ScalarSubcoreMesh(axis_name="c", num_cores=2)

    @functools.partial(
        pl.kernel,
        out_type=[
            jax.ShapeDtypeStruct((1, _PAD_K), jnp.int32),
            jax.ShapeDtypeStruct((B, _TOP_K), jnp.float32),
        ],
        mesh=mesh,
        scratch_types=[
            pltpu.SMEM((B, 1, E), jnp.float32),
            pltpu.SMEM((E,), jnp.float32),
            pltpu.SMEM((1, _PAD_K), jnp.int32),
            pltpu.SMEM((B, _TOP_K), jnp.float32),
            pltpu.SemaphoreType.DMA,
            pltpu.SemaphoreType.DMA,
        ],
    )
    def sel(mv_hbm, idx_hbm, w_hbm, mv_s, gm_s, idx_s, w_s, sem, sem2):
        core = lax.axis_index("c")

        @pl.when(core == 0)
        def _():
            pltpu.async_copy(mv_hbm, mv_s, sem).wait()

            @pl.loop(0, E)
            def _(e):
                acc = mv_s[0, 0, e]
                for b in range(1, B):
                    acc = acc + mv_s[b, 0, e]
                gm_s[e] = acc * (1.0 / B)

            @pl.loop(0, _PAD_K)
            def _(i):
                idx_s[0, i] = 0

            for i in range(_TOP_K):
                def body(e, carry):
                    bv, bi = carry
                    v = gm_s[e]
                    better = v > bv
                    return (
                        jnp.where(better, v, bv),
                        jnp.where(better, e, bi),
                    )

                bv, bi = lax.fori_loop(
                    0, E, body, (jnp.float32(-jnp.inf), jnp.int32(0))
                )
                idx_s[0, i] = bi
                for b in range(B):
                    w_s[b, i] = mv_s[b, 0, bi]
                gm_s[bi] = -jnp.inf

            c1 = pltpu.async_copy(idx_s, idx_hbm, sem)
            c2 = pltpu.async_copy(w_s, w_hbm, sem2)
            c1.wait()
            c2.wait()

    return sel(mean_value)


def _select_body(mv_ref, idx_ref, w_ref):
    B, _, E = mv_ref.shape
    mv = mv_ref[:, 0, :]
    gm = jnp.mean(mv, axis=0, keepdims=True)
    iota_e = lax.broadcasted_iota(jnp.int32, (1, E), 1)
    iota_k = lax.broadcasted_iota(jnp.int32, (1, _PAD_K), 1)
    iota_kb = lax.broadcasted_iota(jnp.int32, (B, _TOP_K), 1)
    idx_row = jnp.zeros((1, _PAD_K), jnp.int32)
    w = jnp.zeros((B, _TOP_K), jnp.float32)
    for i in range(_TOP_K):
        m = jnp.max(gm)
        pos = jnp.min(jnp.where(gm == m, iota_e, E))
        onehot = iota_e == pos
        idx_row = idx_row + pos * (iota_k == i).astype(jnp.int32)
        wcol = jnp.sum(jnp.where(onehot, mv, 0.0), axis=1, keepdims=True)
        w = w + wcol * (iota_kb == i).astype(jnp.float32)
        gm = jnp.where(onehot, -jnp.inf, gm)
    idx_ref[...] = idx_row
    w_ref[...] = w


def _select(mean_value):
    B, _, E = mean_value.shape
    return pl.pallas_call(
        _select_body,
        in_specs=[pl.BlockSpec((B, 1, E), lambda: (0, 0, 0))],
        out_specs=[
            pl.BlockSpec((1, _PAD_K), lambda: (0, 0)),
            pl.BlockSpec((B, _TOP_K), lambda: (0, 0)),
        ],
        out_shape=[
            jax.ShapeDtypeStruct((1, _PAD_K), jnp.int32),
            jax.ShapeDtypeStruct((B, _TOP_K), jnp.float32),
        ],
    )(mean_value)


_WRAP = 128


def _agg_body(idx_ref, w_ref, v_ref, out_ref, scr_ref):
    b = pl.program_id(0)
    L = v_ref.shape[2]

    row = w_ref[pl.ds(b, 1), :]
    row = row - jnp.max(row, axis=1, keepdims=True)
    e = jnp.exp(row)
    sm = e / jnp.sum(e, axis=1, keepdims=True)

    ds = [idx_ref[0, i] for i in range(_TOP_K)]
    TILE = _WRAP
    for j in range(_HB):
        v = v_ref[0, j]
        scr_ref[:_WRAP, :] = v[L - _WRAP :, :]
        scr_ref[_WRAP:, :] = v[:_WRAP, :]
        for t in range((L - _WRAP) // TILE):
            acc = None
            for i in range(_TOP_K):
                term = (
                    v_ref[0, j, pl.ds(ds[i] + t * TILE, TILE), :]
                    * sm[0:1, i : i + 1]
                )
                acc = term if acc is None else acc + term
            out_ref[0, j, t * TILE : (t + 1) * TILE, :] = acc
        tail = None
        for i in range(_TOP_K):
            tterm = scr_ref[pl.ds(ds[i], _WRAP), :] * sm[0:1, i : i + 1]
            tail = tterm if tail is None else tail + tterm
        out_ref[0, j, L - _WRAP :, :] = tail


def _aggregate(values, idx, w):
    B, H, L, E = values.shape
    return pl.pallas_call(
        _agg_body,
        grid=(B, H // _HB),
        in_specs=[
            pl.BlockSpec(memory_space=pltpu.SMEM),
            pl.BlockSpec((B, _TOP_K), lambda b, h: (0, 0)),
            pl.BlockSpec((1, _HB, L, E), lambda b, h: (b, h, 0, 0)),
        ],
        out_specs=pl.BlockSpec((1, _HB, L, E), lambda b, h: (b, h, 0, 0)),
        out_shape=jax.ShapeDtypeStruct((B, H, L, E), jnp.float32),
        scratch_shapes=[pltpu.VMEM((2 * _WRAP, E), jnp.float32)],
    )(idx, w, values)


def kernel(queries, keys, values):
    mv = _mean_value(queries, keys)
    idx, w = _select_sc(mv)
    return _aggregate(values, idx, w)


# speedup vs baseline: 1.4643x; 1.0194x over previous
import dataclasses
import functools
import math

import jax
import jax.numpy as jnp
from jax import lax
from jax.experimental import pallas as pl
from jax.experimental.pallas import tpu as pltpu
from jax.experimental.pallas import tpu_sc as plsc

_TOP_K = 8
_PAD_K = 16


_HB = 4


def _mv_body(q_ref, k_ref, mv_ref, *, scale):
    h = pl.program_id(1)

    @pl.when(h == 0)
    def _():
        mv_ref[...] = jnp.zeros_like(mv_ref)

    acc = None
    for j in range(_HB):
        sq = jnp.sum(q_ref[0, j], axis=0)
        sk = jnp.sum(k_ref[0, j], axis=0)
        t = sq * sk
        acc = t if acc is None else acc + t
    mv_ref[0, 0, :] += acc * scale


def _mean_value(queries, keys):
    B, H, L, E = queries.shape
    return pl.pallas_call(
        functools.partial(_mv_body, scale=1.0 / (H * L)),
        grid=(B, H // _HB),
        in_specs=[
            pl.BlockSpec((1, _HB, L, E), lambda b, h: (b, h, 0, 0)),
            pl.BlockSpec((1, _HB, L, E), lambda b, h: (b, h, 0, 0)),
        ],
        out_specs=pl.BlockSpec((1, 1, E), lambda b, h: (b, 0, 0)),
        out_shape=jax.ShapeDtypeStruct((B, 1, E), jnp.float32),
    )(queries, keys)


_SC_L = 16


def _select_sc_vec(mean_value):
    B, _, E = mean_value.shape
    NCH = E // _SC_L
    mesh = plsc.VectorSubcoreMesh(core_axis_name="c", subcore_axis_name="s")
    cp = pltpu.CompilerParams()
    if "needs_layout_passes" in pltpu.CompilerParams.__dataclass_fields__:
        cp = dataclasses.replace(cp, needs_layout_passes=False)

    @functools.partial(
        pl.kernel,
        out_type=[
            jax.ShapeDtypeStruct((_PAD_K,), jnp.int32),
            jax.ShapeDtypeStruct((_PAD_K,), jnp.float32),
        ],
        mesh=mesh,
        compiler_params=cp,
        scratch_types=[
            pltpu.VMEM((B, 1, E), jnp.float32),
            pltpu.VMEM((_PAD_K,), jnp.int32),
            pltpu.VMEM((_PAD_K,), jnp.float32),
            pltpu.SemaphoreType.DMA,
            pltpu.SemaphoreType.DMA,
        ],
    )
    def sel(mv_hbm, idx_hbm, w_hbm, mv_v, idx_v, w_v, sem, sem2):
        c = lax.axis_index("c")
        s = lax.axis_index("s")

        @pl.when(jnp.logical_and(c == 0, s == 0))
        def _():
            pltpu.async_copy(mv_hbm, mv_v, sem).wait()
            iota = lax.broadcasted_iota(jnp.int32, (_SC_L,), 0)
            chunks = []
            for ch in range(NCH):
                g = None
                for b in range(B):
                    x = mv_v[b, 0, pl.ds(ch * _SC_L, _SC_L)]
                    g = x if g is None else g + x
                chunks.append(g * (1.0 / B))
            idxvec = jnp.zeros((_SC_L,), jnp.int32)
            for i in range(_TOP_K):
                mx = chunks[0]
                for ch in range(1, NCH):
                    mx = jnp.maximum(mx, chunks[ch])
                m = jnp.max(mx)
                cand = None
                for ch in range(NCH):
                    cv = jnp.where(chunks[ch] == m, iota + _SC_L * ch, E)
                    cand = cv if cand is None else jnp.minimum(cand, cv)
                pos = jnp.min(cand)
                idxvec = idxvec + pos * (iota == i).astype(jnp.int32)
                for ch in range(NCH):
                    chunks[ch] = jnp.where(
                        iota + _SC_L * ch == pos, -jnp.inf, chunks[ch]
                    )
            idx_v[...] = idxvec
            bidx = (iota >= _TOP_K).astype(jnp.int32)
            zidx = jnp.zeros((_SC_L,), jnp.int32)
            eidx = plsc.load_gather(idx_v, [iota % _TOP_K])
            w_v[...] = plsc.load_gather(mv_v, [bidx, zidx, eidx])
            c1 = pltpu.async_copy(idx_v, idx_hbm, sem)
            c2 = pltpu.async_copy(w_v, w_hbm, sem2)
            c1.wait()
            c2.wait()

    idx, w = sel(mean_value)
    return idx.reshape(1, _PAD_K), w[: B * _TOP_K].reshape(B, _TOP_K)


def _select_sc(mean_value):
    B, _, E = mean_value.shape
    mesh = plsc.ScalarSubcoreMesh(axis_name="c", num_cores=2)

    @functools.partial(
        pl.kernel,
        out_type=[
            jax.ShapeDtypeStruct((1, _PAD_K), jnp.int32),
            jax.ShapeDtypeStruct((B, _TOP_K), jnp.float32),
        ],
        mesh=mesh,
        scratch_types=[
            pltpu.SMEM((B, 1, E), jnp.float32),
            pltpu.SMEM((E,), jnp.float32),
            pltpu.SMEM((1, _PAD_K), jnp.int32),
            pltpu.SMEM((B, _TOP_K), jnp.float32),
            pltpu.SemaphoreType.DMA,
            pltpu.SemaphoreType.DMA,
        ],
    )
    def sel(mv_hbm, idx_hbm, w_hbm, mv_s, gm_s, idx_s, w_s, sem, sem2):
        core = lax.axis_index("c")

        @pl.when(core == 0)
        def _():
            pltpu.async_copy(mv_hbm, mv_s, sem).wait()

            @pl.loop(0, E)
            def _(e):
                acc = mv_s[0, 0, e]
                for b in range(1, B):
                    acc = acc + mv_s[b, 0, e]
                gm_s[e] = acc * (1.0 / B)

            @pl.loop(0, _PAD_K)
            def _(i):
                idx_s[0, i] = 0

            for i in range(_TOP_K):
                def body(e, carry):
                    bv, bi = carry
                    v = gm_s[e]
                    better = v > bv
                    return (
                        jnp.where(better, v, bv),
                        jnp.where(better, e, bi),
                    )

                bv, bi = lax.fori_loop(
                    0, E, body, (jnp.float32(-jnp.inf), jnp.int32(0))
                )
                idx_s[0, i] = bi
                for b in range(B):
                    w_s[b, i] = mv_s[b, 0, bi]
                gm_s[bi] = -jnp.inf

            c1 = pltpu.async_copy(idx_s, idx_hbm, sem)
            c2 = pltpu.async_copy(w_s, w_hbm, sem2)
            c1.wait()
            c2.wait()

    return sel(mean_value)


def _select_body(mv_ref, idx_ref, w_ref):
    B, _, E = mv_ref.shape
    mv = mv_ref[:, 0, :]
    gm = jnp.mean(mv, axis=0, keepdims=True)
    iota_e = lax.broadcasted_iota(jnp.int32, (1, E), 1)
    iota_k = lax.broadcasted_iota(jnp.int32, (1, _PAD_K), 1)
    iota_kb = lax.broadcasted_iota(jnp.int32, (B, _TOP_K), 1)
    idx_row = jnp.zeros((1, _PAD_K), jnp.int32)
    w = jnp.zeros((B, _TOP_K), jnp.float32)
    for i in range(_TOP_K):
        m = jnp.max(gm)
        pos = jnp.min(jnp.where(gm == m, iota_e, E))
        onehot = iota_e == pos
        idx_row = idx_row + pos * (iota_k == i).astype(jnp.int32)
        wcol = jnp.sum(jnp.where(onehot, mv, 0.0), axis=1, keepdims=True)
        w = w + wcol * (iota_kb == i).astype(jnp.float32)
        gm = jnp.where(onehot, -jnp.inf, gm)
    idx_ref[...] = idx_row
    w_ref[...] = w


def _select(mean_value):
    B, _, E = mean_value.shape
    return pl.pallas_call(
        _select_body,
        in_specs=[pl.BlockSpec((B, 1, E), lambda: (0, 0, 0))],
        out_specs=[
            pl.BlockSpec((1, _PAD_K), lambda: (0, 0)),
            pl.BlockSpec((B, _TOP_K), lambda: (0, 0)),
        ],
        out_shape=[
            jax.ShapeDtypeStruct((1, _PAD_K), jnp.int32),
            jax.ShapeDtypeStruct((B, _TOP_K), jnp.float32),
        ],
    )(mean_value)


_WRAP = 128


def _agg_body(idx_ref, w_ref, v_ref, out_ref, scr_ref):
    b = pl.program_id(0)
    L = v_ref.shape[2]

    row = w_ref[pl.ds(b, 1), :]
    row = row - jnp.max(row, axis=1, keepdims=True)
    e = jnp.exp(row)
    sm = e / jnp.sum(e, axis=1, keepdims=True)

    ds = [idx_ref[0, i] for i in range(_TOP_K)]
    TILE = _WRAP
    for j in range(_HB):
        v = v_ref[0, j]
        scr_ref[:_WRAP, :] = v[L - _WRAP :, :]
        scr_ref[_WRAP:, :] = v[:_WRAP, :]
        for t in range((L - _WRAP) // TILE):
            acc = None
            for i in range(_TOP_K):
                term = (
                    v_ref[0, j, pl.ds(ds[i] + t * TILE, TILE), :]
                    * sm[0:1, i : i + 1]
                )
                acc = term if acc is None else acc + term
            out_ref[0, j, t * TILE : (t + 1) * TILE, :] = acc
        tail = None
        for i in range(_TOP_K):
            tterm = scr_ref[pl.ds(ds[i], _WRAP), :] * sm[0:1, i : i + 1]
            tail = tterm if tail is None else tail + tterm
        out_ref[0, j, L - _WRAP :, :] = tail


def _aggregate(values, idx, w):
    B, H, L, E = values.shape
    return pl.pallas_call(
        _agg_body,
        grid=(B, H // _HB),
        in_specs=[
            pl.BlockSpec(memory_space=pltpu.SMEM),
            pl.BlockSpec((B, _TOP_K), lambda b, h: (0, 0)),
            pl.BlockSpec((1, _HB, L, E), lambda b, h: (b, h, 0, 0)),
        ],
        out_specs=pl.BlockSpec((1, _HB, L, E), lambda b, h: (b, h, 0, 0)),
        out_shape=jax.ShapeDtypeStruct((B, H, L, E), jnp.float32),
        scratch_shapes=[pltpu.VMEM((2 * _WRAP, E), jnp.float32)],
    )(idx, w, values)


def kernel(queries, keys, values):
    mv = _mean_value(queries, keys)
    idx, w = _select_sc_vec(mv)
    return _aggregate(values, idx, w)


# speedup vs baseline: 1.4766x; 1.0084x over previous
import dataclasses
import functools
import math

import jax
import jax.numpy as jnp
from jax import lax
from jax.experimental import pallas as pl
from jax.experimental.pallas import tpu as pltpu
from jax.experimental.pallas import tpu_sc as plsc

_TOP_K = 8
_PAD_K = 16


_HB = 4


def _mv_body(q_ref, k_ref, mv_ref, *, scale):
    h = pl.program_id(1)

    @pl.when(h == 0)
    def _():
        mv_ref[...] = jnp.zeros_like(mv_ref)

    acc = None
    for j in range(_HB):
        sq = jnp.sum(q_ref[0, j], axis=0)
        sk = jnp.sum(k_ref[0, j], axis=0)
        t = sq * sk
        acc = t if acc is None else acc + t
    mv_ref[0, 0, :] += acc * scale


def _mean_value(queries, keys):
    B, H, L, E = queries.shape
    return pl.pallas_call(
        functools.partial(_mv_body, scale=1.0 / (H * L)),
        grid=(B, H // _HB),
        in_specs=[
            pl.BlockSpec((1, _HB, L, E), lambda b, h: (b, h, 0, 0)),
            pl.BlockSpec((1, _HB, L, E), lambda b, h: (b, h, 0, 0)),
        ],
        out_specs=pl.BlockSpec((1, 1, E), lambda b, h: (b, 0, 0)),
        out_shape=jax.ShapeDtypeStruct((B, 1, E), jnp.float32),
    )(queries, keys)


_SC_L = 16


def _select_sc_vec(mean_value):
    B, _, E = mean_value.shape
    NCH = E // _SC_L
    mesh = plsc.VectorSubcoreMesh(core_axis_name="c", subcore_axis_name="s")
    cp = pltpu.CompilerParams()
    if "needs_layout_passes" in pltpu.CompilerParams.__dataclass_fields__:
        cp = dataclasses.replace(cp, needs_layout_passes=False)

    @functools.partial(
        pl.kernel,
        out_type=jax.ShapeDtypeStruct((2, _PAD_K), jnp.int32),
        mesh=mesh,
        compiler_params=cp,
        scratch_types=[
            pltpu.VMEM((B, 1, E), jnp.float32),
            pltpu.VMEM((2, _PAD_K), jnp.int32),
            pltpu.SemaphoreType.DMA,
        ],
    )
    def sel(mv_hbm, pack_hbm, mv_v, pack_v, sem):
        c = lax.axis_index("c")
        s = lax.axis_index("s")

        @pl.when(jnp.logical_and(c == 0, s == 0))
        def _():
            pltpu.async_copy(mv_hbm, mv_v, sem).wait()
            iota = lax.broadcasted_iota(jnp.int32, (_SC_L,), 0)
            chunks = []
            for ch in range(NCH):
                g = None
                for b in range(B):
                    x = mv_v[b, 0, pl.ds(ch * _SC_L, _SC_L)]
                    g = x if g is None else g + x
                chunks.append(g * (1.0 / B))
            idxvec = jnp.zeros((_SC_L,), jnp.int32)
            for i in range(_TOP_K):
                mx = chunks[0]
                for ch in range(1, NCH):
                    mx = jnp.maximum(mx, chunks[ch])
                m = jnp.max(mx)
                cand = None
                for ch in range(NCH):
                    cv = jnp.where(chunks[ch] == m, iota + _SC_L * ch, E)
                    cand = cv if cand is None else jnp.minimum(cand, cv)
                pos = jnp.min(cand)
                idxvec = idxvec + pos * (iota == i).astype(jnp.int32)
                for ch in range(NCH):
                    chunks[ch] = jnp.where(
                        iota + _SC_L * ch == pos, -jnp.inf, chunks[ch]
                    )
            pack_v[0, :] = idxvec
            bidx = (iota >= _TOP_K).astype(jnp.int32)
            zidx = jnp.zeros((_SC_L,), jnp.int32)
            eidx = plsc.load_gather(pack_v, [zidx, iota % _TOP_K])
            wvals = plsc.load_gather(mv_v, [bidx, zidx, eidx])
            pack_v[1, :] = plsc.bitcast(wvals, jnp.int32)
            pltpu.async_copy(pack_v, pack_hbm, sem).wait()

    return sel(mean_value)


def _select_sc(mean_value):
    B, _, E = mean_value.shape
    mesh = plsc.ScalarSubcoreMesh(axis_name="c", num_cores=2)

    @functools.partial(
        pl.kernel,
        out_type=[
            jax.ShapeDtypeStruct((1, _PAD_K), jnp.int32),
            jax.ShapeDtypeStruct((B, _TOP_K), jnp.float32),
        ],
        mesh=mesh,
        scratch_types=[
            pltpu.SMEM((B, 1, E), jnp.float32),
            pltpu.SMEM((E,), jnp.float32),
            pltpu.SMEM((1, _PAD_K), jnp.int32),
            pltpu.SMEM((B, _TOP_K), jnp.float32),
            pltpu.SemaphoreType.DMA,
            pltpu.SemaphoreType.DMA,
        ],
    )
    def sel(mv_hbm, idx_hbm, w_hbm, mv_s, gm_s, idx_s, w_s, sem, sem2):
        core = lax.axis_index("c")

        @pl.when(core == 0)
        def _():
            pltpu.async_copy(mv_hbm, mv_s, sem).wait()

            @pl.loop(0, E)
            def _(e):
                acc = mv_s[0, 0, e]
                for b in range(1, B):
                    acc = acc + mv_s[b, 0, e]
                gm_s[e] = acc * (1.0 / B)

            @pl.loop(0, _PAD_K)
            def _(i):
                idx_s[0, i] = 0

            for i in range(_TOP_K):
                def body(e, carry):
                    bv, bi = carry
                    v = gm_s[e]
                    better = v > bv
                    return (
                        jnp.where(better, v, bv),
                        jnp.where(better, e, bi),
                    )

                bv, bi = lax.fori_loop(
                    0, E, body, (jnp.float32(-jnp.inf), jnp.int32(0))
                )
                idx_s[0, i] = bi
                for b in range(B):
                    w_s[b, i] = mv_s[b, 0, bi]
                gm_s[bi] = -jnp.inf

            c1 = pltpu.async_copy(idx_s, idx_hbm, sem)
            c2 = pltpu.async_copy(w_s, w_hbm, sem2)
            c1.wait()
            c2.wait()

    return sel(mean_value)


def _select_body(mv_ref, idx_ref, w_ref):
    B, _, E = mv_ref.shape
    mv = mv_ref[:, 0, :]
    gm = jnp.mean(mv, axis=0, keepdims=True)
    iota_e = lax.broadcasted_iota(jnp.int32, (1, E), 1)
    iota_k = lax.broadcasted_iota(jnp.int32, (1, _PAD_K), 1)
    iota_kb = lax.broadcasted_iota(jnp.int32, (B, _TOP_K), 1)
    idx_row = jnp.zeros((1, _PAD_K), jnp.int32)
    w = jnp.zeros((B, _TOP_K), jnp.float32)
    for i in range(_TOP_K):
        m = jnp.max(gm)
        pos = jnp.min(jnp.where(gm == m, iota_e, E))
        onehot = iota_e == pos
        idx_row = idx_row + pos * (iota_k == i).astype(jnp.int32)
        wcol = jnp.sum(jnp.where(onehot, mv, 0.0), axis=1, keepdims=True)
        w = w + wcol * (iota_kb == i).astype(jnp.float32)
        gm = jnp.where(onehot, -jnp.inf, gm)
    idx_ref[...] = idx_row
    w_ref[...] = w


def _select(mean_value):
    B, _, E = mean_value.shape
    return pl.pallas_call(
        _select_body,
        in_specs=[pl.BlockSpec((B, 1, E), lambda: (0, 0, 0))],
        out_specs=[
            pl.BlockSpec((1, _PAD_K), lambda: (0, 0)),
            pl.BlockSpec((B, _TOP_K), lambda: (0, 0)),
        ],
        out_shape=[
            jax.ShapeDtypeStruct((1, _PAD_K), jnp.int32),
            jax.ShapeDtypeStruct((B, _TOP_K), jnp.float32),
        ],
    )(mean_value)


_WRAP = 128


def _agg_body(idx_ref, v_ref, out_ref, scr_ref):
    b = pl.program_id(0)
    L = v_ref.shape[2]

    iota8 = lax.broadcasted_iota(jnp.int32, (1, _TOP_K), 1)
    row = jnp.zeros((1, _TOP_K), jnp.float32)
    for i in range(_TOP_K):
        wf = lax.bitcast_convert_type(idx_ref[1, b * _TOP_K + i], jnp.float32)
        row = row + wf * (iota8 == i).astype(jnp.float32)
    row = row - jnp.max(row, axis=1, keepdims=True)
    e = jnp.exp(row)
    sm = e / jnp.sum(e, axis=1, keepdims=True)

    ds = [idx_ref[0, i] for i in range(_TOP_K)]
    TILE = _WRAP
    for j in range(_HB):
        v = v_ref[0, j]
        scr_ref[:_WRAP, :] = v[L - _WRAP :, :]
        scr_ref[_WRAP:, :] = v[:_WRAP, :]
        for t in range((L - _WRAP) // TILE):
            acc = None
            for i in range(_TOP_K):
                term = (
                    v_ref[0, j, pl.ds(ds[i] + t * TILE, TILE), :]
                    * sm[0:1, i : i + 1]
                )
                acc = term if acc is None else acc + term
            out_ref[0, j, t * TILE : (t + 1) * TILE, :] = acc
        tail = None
        for i in range(_TOP_K):
            tterm = scr_ref[pl.ds(ds[i], _WRAP), :] * sm[0:1, i : i + 1]
            tail = tterm if tail is None else tail + tterm
        out_ref[0, j, L - _WRAP :, :] = tail


def _aggregate(values, pack):
    B, H, L, E = values.shape
    return pl.pallas_call(
        _agg_body,
        grid=(B, H // _HB),
        in_specs=[
            pl.BlockSpec(memory_space=pltpu.SMEM),
            pl.BlockSpec((1, _HB, L, E), lambda b, h: (b, h, 0, 0)),
        ],
        out_specs=pl.BlockSpec((1, _HB, L, E), lambda b, h: (b, h, 0, 0)),
        out_shape=jax.ShapeDtypeStruct((B, H, L, E), jnp.float32),
        scratch_shapes=[pltpu.VMEM((2 * _WRAP, E), jnp.float32)],
    )(pack, values)


def kernel(queries, keys, values):
    mv = _mean_value(queries, keys)
    pack = _select_sc_vec(mv)
    return _aggregate(values, pack)


# speedup vs baseline: 1.4789x; 1.0016x over previous
import dataclasses
import functools

import jax
import jax.numpy as jnp
from jax import lax
from jax.experimental import pallas as pl
from jax.experimental.pallas import tpu as pltpu
from jax.experimental.pallas import tpu_sc as plsc

_TOP_K = 8
_PAD_K = 16


_HB = 4


def _mv_body(q_ref, k_ref, mv_ref, *, scale):
    h = pl.program_id(1)

    @pl.when(h == 0)
    def _():
        mv_ref[...] = jnp.zeros_like(mv_ref)

    acc = None
    for j in range(_HB):
        sq = jnp.sum(q_ref[0, j], axis=0)
        sk = jnp.sum(k_ref[0, j], axis=0)
        t = sq * sk
        acc = t if acc is None else acc + t
    mv_ref[0, 0, :] += acc * scale


def _mean_value(queries, keys):
    B, H, L, E = queries.shape
    return pl.pallas_call(
        functools.partial(_mv_body, scale=1.0 / (H * L)),
        grid=(B, H // _HB),
        in_specs=[
            pl.BlockSpec((1, _HB, L, E), lambda b, h: (b, h, 0, 0)),
            pl.BlockSpec((1, _HB, L, E), lambda b, h: (b, h, 0, 0)),
        ],
        out_specs=pl.BlockSpec((1, 1, E), lambda b, h: (b, 0, 0)),
        out_shape=jax.ShapeDtypeStruct((B, 1, E), jnp.float32),
    )(queries, keys)


_SC_L = 16


def _select_sc_vec(mean_value):
    B, _, E = mean_value.shape
    NCH = E // _SC_L
    mesh = plsc.VectorSubcoreMesh(core_axis_name="c", subcore_axis_name="s")
    cp = pltpu.CompilerParams()
    if "needs_layout_passes" in pltpu.CompilerParams.__dataclass_fields__:
        cp = dataclasses.replace(cp, needs_layout_passes=False)

    @functools.partial(
        pl.kernel,
        out_type=jax.ShapeDtypeStruct((2, _PAD_K), jnp.int32),
        mesh=mesh,
        compiler_params=cp,
        scratch_types=[
            pltpu.VMEM((B, 1, E), jnp.float32),
            pltpu.VMEM((2, _PAD_K), jnp.int32),
            pltpu.SemaphoreType.DMA,
        ],
    )
    def sel(mv_hbm, pack_hbm, mv_v, pack_v, sem):
        c = lax.axis_index("c")
        s = lax.axis_index("s")

        @pl.when(jnp.logical_and(c == 0, s == 0))
        def _():
            pltpu.async_copy(mv_hbm, mv_v, sem).wait()
            iota = lax.broadcasted_iota(jnp.int32, (_SC_L,), 0)
            chunks = []
            for ch in range(NCH):
                g = None
                for b in range(B):
                    x = mv_v[b, 0, pl.ds(ch * _SC_L, _SC_L)]
                    g = x if g is None else g + x
                chunks.append(g * (1.0 / B))
            idxvec = jnp.zeros((_SC_L,), jnp.int32)
            for i in range(_TOP_K):
                mx = chunks[0]
                for ch in range(1, NCH):
                    mx = jnp.maximum(mx, chunks[ch])
                m = jnp.max(mx)
                cand = None
                for ch in range(NCH):
                    cv = jnp.where(chunks[ch] == m, iota + _SC_L * ch, E)
                    cand = cv if cand is None else jnp.minimum(cand, cv)
                pos = jnp.min(cand)
                idxvec = idxvec + pos * (iota == i).astype(jnp.int32)
                for ch in range(NCH):
                    chunks[ch] = jnp.where(
                        iota + _SC_L * ch == pos, -jnp.inf, chunks[ch]
                    )
            pack_v[0, :] = idxvec
            bidx = (iota >= _TOP_K).astype(jnp.int32)
            zidx = jnp.zeros((_SC_L,), jnp.int32)
            eidx = plsc.load_gather(pack_v, [zidx, iota % _TOP_K])
            wvals = plsc.load_gather(mv_v, [bidx, zidx, eidx])
            pack_v[1, :] = plsc.bitcast(wvals, jnp.int32)
            pltpu.async_copy(pack_v, pack_hbm, sem).wait()

    return sel(mean_value)


_WRAP = 128


def _agg_body(idx_ref, v_ref, out_ref, scr_ref):
    b = pl.program_id(0)
    L = v_ref.shape[2]

    iota8 = lax.broadcasted_iota(jnp.int32, (1, _TOP_K), 1)
    row = jnp.zeros((1, _TOP_K), jnp.float32)
    for i in range(_TOP_K):
        wf = lax.bitcast_convert_type(idx_ref[1, b * _TOP_K + i], jnp.float32)
        row = row + wf * (iota8 == i).astype(jnp.float32)
    row = row - jnp.max(row, axis=1, keepdims=True)
    e = jnp.exp(row)
    sm = e / jnp.sum(e, axis=1, keepdims=True)

    ds = [idx_ref[0, i] for i in range(_TOP_K)]
    TILE = _WRAP
    for j in range(_HB):
        v = v_ref[0, j]
        scr_ref[:_WRAP, :] = v[L - _WRAP :, :]
        scr_ref[_WRAP:, :] = v[:_WRAP, :]
        for t in range((L - _WRAP) // TILE):
            acc = None
            for i in range(_TOP_K):
                term = (
                    v_ref[0, j, pl.ds(ds[i] + t * TILE, TILE), :]
                    * sm[0:1, i : i + 1]
                )
                acc = term if acc is None else acc + term
            out_ref[0, j, t * TILE : (t + 1) * TILE, :] = acc
        tail = None
        for i in range(_TOP_K):
            tterm = scr_ref[pl.ds(ds[i], _WRAP), :] * sm[0:1, i : i + 1]
            tail = tterm if tail is None else tail + tterm
        out_ref[0, j, L - _WRAP :, :] = tail


def _aggregate(values, pack):
    B, H, L, E = values.shape
    return pl.pallas_call(
        _agg_body,
        grid=(B, H // _HB),
        in_specs=[
            pl.BlockSpec(memory_space=pltpu.SMEM),
            pl.BlockSpec((1, _HB, L, E), lambda b, h: (b, h, 0, 0)),
        ],
        out_specs=pl.BlockSpec((1, _HB, L, E), lambda b, h: (b, h, 0, 0)),
        out_shape=jax.ShapeDtypeStruct((B, H, L, E), jnp.float32),
        scratch_shapes=[pltpu.VMEM((2 * _WRAP, E), jnp.float32)],
    )(pack, values)


def kernel(queries, keys, values):
    mv = _mean_value(queries, keys)
    pack = _select_sc_vec(mv)
    return _aggregate(values, pack)
